# Initial kernel scaffold; baseline (speedup 1.0000x reference)
import jax
import jax.numpy as jnp
from jax.experimental import pallas as pl


def kernel(x, edge_index, edge_attr, global_attr, W10, b10, W11, b11, W12, b12, W13, b13, W20, b20, W21, b21, W22, b22, W23, b23):
    raise NotImplementedError("write your pallas kernel here")


# speedup vs baseline: 3.2758x; 3.2758x over previous
import functools

import jax
import jax.numpy as jnp
from jax import lax
from jax.experimental import pallas as pl
from jax.experimental.pallas import tpu as pltpu
from jax.experimental.pallas import tpu_sc as plsc

_N = 10000
_E = 320000
_D = 128

_NC = 2
_NS = 16
_NW = _NC * _NS
_PER_W = _E // _NW
_CH = 400
_N_PAD = 10240
_N_HALF = _N_PAD // _NC
_PER_T = _E // _NS
_ROWS_T = _N_HALF // _NS


def _sc_gather_count(x, row, col):
    mesh = plsc.VectorSubcoreMesh(core_axis_name="c", subcore_axis_name="s")

    @functools.partial(
        pl.kernel,
        out_type=(
            jax.ShapeDtypeStruct((_E, _D), jnp.float32),
            jax.ShapeDtypeStruct((_NW * _N_PAD,), jnp.float32),
        ),
        mesh=mesh,
        scratch_types=[
            pltpu.VMEM((_CH,), jnp.int32),
            pltpu.VMEM((_CH,), jnp.int32),
            pltpu.VMEM((_CH, _D), jnp.float32),
            pltpu.VMEM((_N_PAD,), jnp.float32),
            pltpu.SemaphoreType.DMA,
        ],
        compiler_params=pltpu.CompilerParams(needs_layout_passes=False),
    )
    def k(x_hbm, row_hbm, col_hbm, out_hbm, cnt_hbm, idx_v, col_v, rows_v,
          hist_v, sem):
        wid = lax.axis_index("s") * _NC + lax.axis_index("c")
        zeros16 = jnp.zeros((16,), jnp.float32)
        ones16 = jnp.ones((16,), jnp.float32)

        def zbody(i, carry):
            hist_v[pl.ds(i * 16, 16)] = zeros16
            return carry

        lax.fori_loop(0, _N_PAD // 16, zbody, 0)

        def body(i, carry):
            base = wid * _PER_W + i * _CH
            pltpu.sync_copy(row_hbm.at[pl.ds(base, _CH)], idx_v)
            pltpu.sync_copy(col_hbm.at[pl.ds(base, _CH)], col_v)
            gat = pltpu.async_copy(x_hbm.at[idx_v], rows_v, sem)

            def hbody(j, c2):
                idx16 = col_v[pl.ds(j * 16, 16)]
                plsc.addupdate_scatter(hist_v, [idx16], ones16)
                return c2

            lax.fori_loop(0, _CH // 16, hbody, 0)
            gat.wait()
            pltpu.sync_copy(rows_v, out_hbm.at[pl.ds(base, _CH)])
            return carry

        lax.fori_loop(0, _PER_W // _CH, body, 0)
        pltpu.sync_copy(hist_v, cnt_hbm.at[pl.ds(wid * _N_PAD, _N_PAD)])

    return k(x, row, col)


def _sc_scatter(h, col, zeros_tile):
    mesh = plsc.VectorSubcoreMesh(core_axis_name="c", subcore_axis_name="s")

    @functools.partial(
        pl.kernel,
        out_type=jax.ShapeDtypeStruct((_NC, _N_HALF, _D), jnp.float32),
        mesh=mesh,
        scratch_types=[
            pltpu.VMEM((_CH,), jnp.int32),
            pltpu.VMEM((_CH, _D), jnp.float32),
            pltpu.VMEM_SHARED((_N_HALF + 8, _D), jnp.float32),
            pltpu.SemaphoreType.DMA,
        ],
        compiler_params=pltpu.CompilerParams(needs_layout_passes=False),
    )
    def k(h_hbm, col_hbm, zero_hbm, out_hbm, idx_v, rows_v, acc_sh, sem):
        c = lax.axis_index("c")
        s = lax.axis_index("s")
        nbase = c * _N_HALF
        pltpu.sync_copy(zero_hbm, acc_sh.at[pl.ds(s * _ROWS_T, _ROWS_T)])

        @pl.when(s == 0)
        def _():
            pltpu.sync_copy(zero_hbm.at[pl.ds(0, 8)],
                            acc_sh.at[pl.ds(_N_HALF, 8)])

        plsc.subcore_barrier()

        def body(i, carry):
            base = s * _PER_T + i * _CH
            pltpu.sync_copy(col_hbm.at[pl.ds(base, _CH)], idx_v)
            pltpu.sync_copy(h_hbm.at[pl.ds(base, _CH)], rows_v)

            def tbody(j, c2):
                c16 = idx_v[pl.ds(j * 16, 16)]
                loc = c16 - nbase
                ok = jnp.logical_and(loc >= 0, loc < _N_HALF)
                idx_v[pl.ds(j * 16, 16)] = jnp.where(ok, loc, _N_HALF)
                return c2

            lax.fori_loop(0, _CH // 16, tbody, 0)
            pltpu.sync_copy(rows_v, acc_sh.at[idx_v], add=True)
            return carry

        lax.fori_loop(0, _PER_T // _CH, body, 0)
        plsc.subcore_barrier()
        pltpu.sync_copy(
            acc_sh.at[pl.ds(s * _ROWS_T, _ROWS_T)],
            out_hbm.at[c].at[pl.ds(s * _ROWS_T, _ROWS_T)],
        )

    return k(h, col, zeros_tile)


def _edge_mlp(g, ea, w0x, w0e, b0, w1, b1, w2, b2, w3, b3, w4):
    BE = 2560
    grid = (_E // BE,)

    def body(g_ref, e_ref, w0x_r, w0e_r, b0_r, w1_r, b1_r, w2_r, b2_r,
             w3_r, b3_r, w4_r, out_ref):
        h = g_ref[...] @ w0x_r[...] + e_ref[...] @ w0e_r[...] + b0_r[...]
        h = jnp.maximum(h, 0.0)
        h = jnp.maximum(h @ w1_r[...] + b1_r[...], 0.0)
        h = jnp.maximum(h @ w2_r[...] + b2_r[...], 0.0)
        h = jnp.maximum(h @ w3_r[...] + b3_r[...], 0.0)
        out_ref[...] = h @ w4_r[...]

    full = lambda shape: pl.BlockSpec(shape, lambda i: (0,) * len(shape))
    return pl.pallas_call(
        body,
        grid=grid,
        in_specs=[
            pl.BlockSpec((BE, _D), lambda i: (i, 0)),
            pl.BlockSpec((BE, 16), lambda i: (i, 0)),
            full((_D, 128)), full((16, 128)), full((1, 128)),
            full((128, 128)), full((1, 128)),
            full((128, 128)), full((1, 128)),
            full((128, 144)), full((1, 144)),
            full((144, 128)),
        ],
        out_specs=pl.BlockSpec((BE, _D), lambda i: (i, 0)),
        out_shape=jax.ShapeDtypeStruct((_E, _D), jnp.float32),
    )(g, ea, w0x, w0e, b0, w1, b1, w2, b2, w3, b3, w4)


def _node_mlp(x, partials, cnt_parts, ga, w0x, w0g, b0, w1, b1, w2, b2,
              w3, b3):
    BN = 2048
    grid = (_N_PAD // BN,)

    def body(x_ref, p_ref, c_ref, ones_r, ga_r, w0x_r, w0g_r, b0_r, w1_r,
             b1_r, w2_r, b2_r, w3_r, b3_r, out_ref):
        s = p_ref[...]
        cnt = lax.dot_general(c_ref[...], ones_r[...],
                              (((0,), (0,)), ((), ())))
        agg = s * (1.0 / jnp.maximum(cnt, 1.0))
        gb = ga_r[...] @ w0g_r[...] + b0_r[...]
        h = jnp.maximum(x_ref[...] @ w0x_r[...] + agg + gb, 0.0)
        h = jnp.maximum(h @ w1_r[...] + b1_r[...], 0.0)
        h = jnp.maximum(h @ w2_r[...] + b2_r[...], 0.0)
        out_ref[...] = h @ w3_r[...] + b3_r[...]

    full = lambda shape: pl.BlockSpec(shape, lambda i: (0,) * len(shape))
    return pl.pallas_call(
        body,
        grid=grid,
        in_specs=[
            pl.BlockSpec((BN, _D), lambda i: (i, 0)),
            pl.BlockSpec((BN, _D), lambda i: (i, 0)),
            pl.BlockSpec((_NW, BN), lambda i: (0, i)),
            full((_NW, 1)),
            full((1, 16)),
            full((_D, 128)), full((16, 128)), full((1, 128)),
            full((128, 128)), full((1, 128)),
            full((128, 128)), full((1, 128)),
            full((128, 128)), full((1, 128)),
        ],
        out_specs=pl.BlockSpec((BN, 128), lambda i: (i, 0)),
        out_shape=jax.ShapeDtypeStruct((_N_PAD, 128), jnp.float32),
    )(x, partials, cnt_parts, jnp.ones((_NW, 1), jnp.float32), ga, w0x,
      w0g, b0, w1, b1, w2, b2, w3, b3)


def kernel(x, edge_index, edge_attr, global_attr,
           W10, b10, W11, b11, W12, b12, W13, b13,
           W20, b20, W21, b21, W22, b22, W23, b23):
    row = edge_index[0]
    col = edge_index[1]

    w10x = W10[:_D]
    w10e = W10[_D:]
    w20x = W20[:_D]
    w20a = W20[_D:_D + 144]
    w20g = W20[_D + 144:]

    gathered, cnt_flat = _sc_gather_count(x, row, col)
    cnt_parts = cnt_flat.reshape(_NW, _N_PAD)
    h5 = _edge_mlp(gathered, edge_attr,
                   w10x, w10e, b10.reshape(1, -1),
                   W11, b11.reshape(1, -1),
                   W12, b12.reshape(1, -1),
                   W13, b13.reshape(1, -1),
                   w20a)
    zeros_tile = jnp.zeros((_ROWS_T, _D), jnp.float32)
    sums = _sc_scatter(h5, col, zeros_tile).reshape(_N_PAD, _D)
    x_pad = jnp.concatenate(
        [x, jnp.zeros((_N_PAD - _N, _D), jnp.float32)], axis=0)
    out = _node_mlp(x_pad, sums, cnt_parts, global_attr,
                    w20x, w20g, b20.reshape(1, -1),
                    W21, b21.reshape(1, -1),
                    W22, b22.reshape(1, -1),
                    W23, b23.reshape(1, -1))
    return out[:_N]


# speedup vs baseline: 3.5480x; 1.0831x over previous
import functools

import jax
import jax.numpy as jnp
from jax import lax
from jax.experimental import pallas as pl
from jax.experimental.pallas import tpu as pltpu
from jax.experimental.pallas import tpu_sc as plsc

_N = 10000
_E = 320000
_D = 128

_NC = 2
_NS = 16
_NW = _NC * _NS
_PER_W = _E // _NW
_CH = 400
_CHA = 256
_CHB = 144
_N_PAD = 10240
_N_HALF = _N_PAD // _NC
_PER_T = _E // _NS
_ROWS_T = _N_HALF // _NS


def _sc_gather_count(x, row, col):
    mesh = plsc.VectorSubcoreMesh(core_axis_name="c", subcore_axis_name="s")
    CHG = 200
    NCH = _PER_W // CHG

    @functools.partial(
        pl.kernel,
        out_type=(
            jax.ShapeDtypeStruct((_E, _D), jnp.float32),
            jax.ShapeDtypeStruct((_NW * _N_PAD,), jnp.float32),
        ),
        mesh=mesh,
        scratch_types=[
            pltpu.VMEM((_PER_W,), jnp.int32),
            pltpu.VMEM((_PER_W,), jnp.int32),
            pltpu.VMEM((CHG, _D), jnp.float32),
            pltpu.VMEM((CHG, _D), jnp.float32),
            pltpu.VMEM((_N_PAD,), jnp.float32),
            pltpu.SemaphoreType.DMA,
            pltpu.SemaphoreType.DMA,
            pltpu.SemaphoreType.DMA,
            pltpu.SemaphoreType.DMA,
        ],
        compiler_params=pltpu.CompilerParams(needs_layout_passes=False),
    )
    def k(x_hbm, row_hbm, col_hbm, out_hbm, cnt_hbm, idx_v, col_v, rows_a,
          rows_b, hist_v, sem_i, sem_g, sem_wa, sem_wb):
        wid = lax.axis_index("s") * _NC + lax.axis_index("c")
        span = wid * _PER_W
        zeros16 = jnp.zeros((16,), jnp.float32)
        ones16 = jnp.ones((16,), jnp.float32)

        cp_i = pltpu.async_copy(row_hbm.at[pl.ds(span, _PER_W)], idx_v, sem_i)
        cp_c = pltpu.async_copy(col_hbm.at[pl.ds(span, _PER_W)], col_v, sem_i)

        def zbody(i, carry):
            hist_v[pl.ds(i * 16, 16)] = zeros16
            return carry

        lax.fori_loop(0, _N_PAD // 16, zbody, 0)
        cp_i.wait()
        cp_c.wait()

        bufs = ((rows_a, sem_wa), (rows_b, sem_wb))

        def body(io, carry):
            for b, (rows_v, sem_w) in enumerate(bufs):
                ci = io * 2 + b

                @pl.when(io > 0)
                def _():
                    old = span + (ci - 2) * CHG
                    pltpu.make_async_copy(
                        rows_v, out_hbm.at[pl.ds(old, CHG)], sem_w).wait()

                pltpu.async_copy(
                    x_hbm.at[idx_v.at[pl.ds(ci * CHG, CHG)]], rows_v,
                    sem_g).wait()
                pltpu.async_copy(
                    rows_v, out_hbm.at[pl.ds(span + ci * CHG, CHG)], sem_w)
            return carry

        lax.fori_loop(0, NCH // 2, body, 0)
        for b, (rows_v, sem_w) in enumerate(bufs):
            old = span + (NCH - 2 + b) * CHG
            pltpu.make_async_copy(
                rows_v, out_hbm.at[pl.ds(old, CHG)], sem_w).wait()

        def hbody(j, c2):
            idx16 = col_v[pl.ds(j * 16, 16)]
            plsc.addupdate_scatter(hist_v, [idx16], ones16)
            return c2

        lax.fori_loop(0, _PER_W // 16, hbody, 0)
        pltpu.sync_copy(hist_v, cnt_hbm.at[pl.ds(wid * _N_PAD, _N_PAD)])

    return k(x, row, col)


def _sc_scatter(h, col, zeros_tile):
    mesh = plsc.VectorSubcoreMesh(core_axis_name="c", subcore_axis_name="s")

    @functools.partial(
        pl.kernel,
        out_type=jax.ShapeDtypeStruct((_NC, _N_HALF, _D), jnp.float32),
        mesh=mesh,
        scratch_types=[
            pltpu.VMEM((_CHA,), jnp.int32),
            pltpu.VMEM((_CHB,), jnp.int32),
            pltpu.VMEM((_CHA, _D), jnp.float32),
            pltpu.VMEM((_CHB, _D), jnp.float32),
            pltpu.VMEM_SHARED((_N_HALF + 8, _D), jnp.float32),
            pltpu.SemaphoreType.DMA,
            pltpu.SemaphoreType.DMA,
            pltpu.SemaphoreType.DMA,
            pltpu.SemaphoreType.DMA,
            pltpu.SemaphoreType.DMA,
            pltpu.SemaphoreType.DMA,
        ],
        compiler_params=pltpu.CompilerParams(needs_layout_passes=False),
    )
    def k(h_hbm, col_hbm, zero_hbm, out_hbm, idx_a, idx_b, rows_a, rows_b,
          acc_sh, sem_ia, sem_ib, sem_ra, sem_rb, sem_sa, sem_sb):
        c = lax.axis_index("c")
        s = lax.axis_index("s")
        nbase = c * _N_HALF
        NPAIR = _PER_T // (_CHA + _CHB)
        bufs = ((idx_a, rows_a, _CHA, 0, sem_ia, sem_ra, sem_sa),
                (idx_b, rows_b, _CHB, _CHA, sem_ib, sem_rb, sem_sb))

        def start_dma(pair, bset):
            idx_v, rows_v, ch, off, sem_i, sem_r, _ = bset
            base = s * _PER_T + pair * (_CHA + _CHB) + off
            pltpu.async_copy(col_hbm.at[pl.ds(base, ch)], idx_v, sem_i)
            pltpu.async_copy(h_hbm.at[pl.ds(base, ch)], rows_v, sem_r)

        def wait_dma(pair, bset):
            idx_v, rows_v, ch, off, sem_i, sem_r, _ = bset
            base = s * _PER_T + pair * (_CHA + _CHB) + off
            pltpu.make_async_copy(
                col_hbm.at[pl.ds(base, ch)], idx_v, sem_i).wait()
            pltpu.make_async_copy(
                h_hbm.at[pl.ds(base, ch)], rows_v, sem_r).wait()

        def wait_stream(bset):
            idx_v, rows_v = bset[0], bset[1]
            pltpu.make_async_copy(rows_v, acc_sh.at[idx_v], bset[6]).wait()

        start_dma(0, bufs[0])
        pltpu.sync_copy(zero_hbm, acc_sh.at[pl.ds(s * _ROWS_T, _ROWS_T)])

        @pl.when(s == 0)
        def _():
            pltpu.sync_copy(zero_hbm.at[pl.ds(0, 8)],
                            acc_sh.at[pl.ds(_N_HALF, 8)])

        plsc.subcore_barrier()

        def body(io, carry):
            for b in range(2):
                bset = bufs[b]
                idx_v, rows_v, ch = bset[0], bset[1], bset[2]
                sem_s = bset[6]
                wait_dma(io, bset)

                def tbody(j, c2):
                    c16 = idx_v[pl.ds(j * 16, 16)]
                    loc = c16 - nbase
                    ok = jnp.logical_and(loc >= 0, loc < _N_HALF)
                    idx_v[pl.ds(j * 16, 16)] = jnp.where(ok, loc, _N_HALF)
                    return c2

                lax.fori_loop(0, ch // 16, tbody, 0)
                pltpu.async_copy(rows_v, acc_sh.at[idx_v], sem_s, add=True)
                other = bufs[1 - b]
                if b == 0:
                    @pl.when(io >= 1)
                    def _():
                        wait_stream(other)

                    start_dma(io, other)
                else:
                    @pl.when(io < NPAIR - 1)
                    def _():
                        wait_stream(other)
                        start_dma(io + 1, other)

            return carry

        lax.fori_loop(0, NPAIR, body, 0)
        wait_stream(bufs[0])
        wait_stream(bufs[1])
        plsc.subcore_barrier()
        pltpu.sync_copy(
            acc_sh.at[pl.ds(s * _ROWS_T, _ROWS_T)],
            out_hbm.at[c].at[pl.ds(s * _ROWS_T, _ROWS_T)],
        )

    return k(h, col, zeros_tile)


def _edge_mlp(g, ea, w0x, w0e, b0, w1, b1, w2, b2, w3, b3, w4):
    BE = 2560
    grid = (_E // BE,)

    def body(g_ref, e_ref, w0x_r, w0e_r, b0_r, w1_r, b1_r, w2_r, b2_r,
             w3_r, b3_r, w4_r, out_ref):
        h = g_ref[...] @ w0x_r[...] + e_ref[...] @ w0e_r[...] + b0_r[...]
        h = jnp.maximum(h, 0.0)
        h = jnp.maximum(h @ w1_r[...] + b1_r[...], 0.0)
        h = jnp.maximum(h @ w2_r[...] + b2_r[...], 0.0)
        h = jnp.maximum(h @ w3_r[...] + b3_r[...], 0.0)
        out_ref[...] = h @ w4_r[...]

    full = lambda shape: pl.BlockSpec(shape, lambda i: (0,) * len(shape))
    return pl.pallas_call(
        body,
        grid=grid,
        in_specs=[
            pl.BlockSpec((BE, _D), lambda i: (i, 0)),
            pl.BlockSpec((BE, 16), lambda i: (i, 0)),
            full((_D, 128)), full((16, 128)), full((1, 128)),
            full((128, 128)), full((1, 128)),
            full((128, 128)), full((1, 128)),
            full((128, 144)), full((1, 144)),
            full((144, 128)),
        ],
        out_specs=pl.BlockSpec((BE, _D), lambda i: (i, 0)),
        out_shape=jax.ShapeDtypeStruct((_E, _D), jnp.float32),
    )(g, ea, w0x, w0e, b0, w1, b1, w2, b2, w3, b3, w4)


def _node_mlp(x, partials, cnt_parts, ga, w0x, w0g, b0, w1, b1, w2, b2,
              w3, b3):
    BN = 2048
    grid = (_N_PAD // BN,)

    def body(x_ref, p_ref, c_ref, ones_r, ga_r, w0x_r, w0g_r, b0_r, w1_r,
             b1_r, w2_r, b2_r, w3_r, b3_r, out_ref):
        s = p_ref[...]
        cnt = lax.dot_general(c_ref[...], ones_r[...],
                              (((0,), (0,)), ((), ())))
        agg = s * (1.0 / jnp.maximum(cnt, 1.0))
        gb = ga_r[...] @ w0g_r[...] + b0_r[...]
        h = jnp.maximum(x_ref[...] @ w0x_r[...] + agg + gb, 0.0)
        h = jnp.maximum(h @ w1_r[...] + b1_r[...], 0.0)
        h = jnp.maximum(h @ w2_r[...] + b2_r[...], 0.0)
        out_ref[...] = h @ w3_r[...] + b3_r[...]

    full = lambda shape: pl.BlockSpec(shape, lambda i: (0,) * len(shape))
    return pl.pallas_call(
        body,
        grid=grid,
        in_specs=[
            pl.BlockSpec((BN, _D), lambda i: (i, 0)),
            pl.BlockSpec((BN, _D), lambda i: (i, 0)),
            pl.BlockSpec((_NW, BN), lambda i: (0, i)),
            full((_NW, 1)),
            full((1, 16)),
            full((_D, 128)), full((16, 128)), full((1, 128)),
            full((128, 128)), full((1, 128)),
            full((128, 128)), full((1, 128)),
            full((128, 128)), full((1, 128)),
        ],
        out_specs=pl.BlockSpec((BN, 128), lambda i: (i, 0)),
        out_shape=jax.ShapeDtypeStruct((_N_PAD, 128), jnp.float32),
    )(x, partials, cnt_parts, jnp.ones((_NW, 1), jnp.float32), ga, w0x,
      w0g, b0, w1, b1, w2, b2, w3, b3)


def kernel(x, edge_index, edge_attr, global_attr,
           W10, b10, W11, b11, W12, b12, W13, b13,
           W20, b20, W21, b21, W22, b22, W23, b23):
    row = edge_index[0]
    col = edge_index[1]

    w10x = W10[:_D]
    w10e = W10[_D:]
    w20x = W20[:_D]
    w20a = W20[_D:_D + 144]
    w20g = W20[_D + 144:]

    gathered, cnt_flat = _sc_gather_count(x, row, col)
    cnt_parts = cnt_flat.reshape(_NW, _N_PAD)
    h5 = _edge_mlp(gathered, edge_attr,
                   w10x, w10e, b10.reshape(1, -1),
                   W11, b11.reshape(1, -1),
                   W12, b12.reshape(1, -1),
                   W13, b13.reshape(1, -1),
                   w20a)
    zeros_tile = jnp.zeros((_ROWS_T, _D), jnp.float32)
    sums = _sc_scatter(h5, col, zeros_tile).reshape(_N_PAD, _D)
    x_pad = jnp.concatenate(
        [x, jnp.zeros((_N_PAD - _N, _D), jnp.float32)], axis=0)
    out = _node_mlp(x_pad, sums, cnt_parts, global_attr,
                    w20x, w20g, b20.reshape(1, -1),
                    W21, b21.reshape(1, -1),
                    W22, b22.reshape(1, -1),
                    W23, b23.reshape(1, -1))
    return out[:_N]


# speedup vs baseline: 3.7090x; 1.0454x over previous
import functools

import jax
import jax.numpy as jnp
from jax import lax
from jax.experimental import pallas as pl
from jax.experimental.pallas import tpu as pltpu
from jax.experimental.pallas import tpu_sc as plsc

_N = 10000
_E = 320000
_D = 128

_NC = 2
_NS = 16
_NW = _NC * _NS
_PER_W = _E // _NW
_CH = 400
_DP = 64
_CHA = 256
_CHB = 144
_N_PAD = 10240
_N_HALF = _N_PAD // _NC
_PER_T = _E // _NS
_ROWS_T = _N_HALF // _NS


def _sc_gather_count(x, row, col):
    mesh = plsc.VectorSubcoreMesh(core_axis_name="c", subcore_axis_name="s")
    CHG = 200
    NCH = _PER_W // CHG

    @functools.partial(
        pl.kernel,
        out_type=(
            jax.ShapeDtypeStruct((_E, _D), jnp.float32),
            jax.ShapeDtypeStruct((_NW * _N_PAD,), jnp.float32),
        ),
        mesh=mesh,
        scratch_types=[
            pltpu.VMEM((_PER_W,), jnp.int32),
            pltpu.VMEM((_PER_W,), jnp.int32),
            pltpu.VMEM((CHG, _D), jnp.float32),
            pltpu.VMEM((CHG, _D), jnp.float32),
            pltpu.VMEM((_N_PAD,), jnp.float32),
            pltpu.SemaphoreType.DMA,
            pltpu.SemaphoreType.DMA,
            pltpu.SemaphoreType.DMA,
            pltpu.SemaphoreType.DMA,
        ],
        compiler_params=pltpu.CompilerParams(needs_layout_passes=False),
    )
    def k(x_hbm, row_hbm, col_hbm, out_hbm, cnt_hbm, idx_v, col_v, rows_a,
          rows_b, hist_v, sem_i, sem_g, sem_wa, sem_wb):
        wid = lax.axis_index("s") * _NC + lax.axis_index("c")
        span = wid * _PER_W
        zeros16 = jnp.zeros((16,), jnp.float32)
        ones16 = jnp.ones((16,), jnp.float32)

        cp_i = pltpu.async_copy(row_hbm.at[pl.ds(span, _PER_W)], idx_v, sem_i)
        cp_c = pltpu.async_copy(col_hbm.at[pl.ds(span, _PER_W)], col_v, sem_i)

        def zbody(i, carry):
            hist_v[pl.ds(i * 16, 16)] = zeros16
            return carry

        lax.fori_loop(0, _N_PAD // 16, zbody, 0)
        cp_i.wait()
        cp_c.wait()

        bufs = ((rows_a, sem_wa), (rows_b, sem_wb))

        def body(io, carry):
            for b, (rows_v, sem_w) in enumerate(bufs):
                ci = io * 2 + b

                @pl.when(io > 0)
                def _():
                    old = span + (ci - 2) * CHG
                    pltpu.make_async_copy(
                        rows_v, out_hbm.at[pl.ds(old, CHG)], sem_w).wait()

                pltpu.async_copy(
                    x_hbm.at[idx_v.at[pl.ds(ci * CHG, CHG)]], rows_v,
                    sem_g).wait()
                pltpu.async_copy(
                    rows_v, out_hbm.at[pl.ds(span + ci * CHG, CHG)], sem_w)
            return carry

        lax.fori_loop(0, NCH // 2, body, 0)
        for b, (rows_v, sem_w) in enumerate(bufs):
            old = span + (NCH - 2 + b) * CHG
            pltpu.make_async_copy(
                rows_v, out_hbm.at[pl.ds(old, CHG)], sem_w).wait()

        def hbody(j, c2):
            idx16 = col_v[pl.ds(j * 16, 16)]
            plsc.addupdate_scatter(hist_v, [idx16], ones16)
            return c2

        lax.fori_loop(0, _PER_W // 16, hbody, 0)
        pltpu.sync_copy(hist_v, cnt_hbm.at[pl.ds(wid * _N_PAD, _N_PAD)])

    return k(x, row, col)


def _sc_scatter(h, col, zeros_tile):
    mesh = plsc.VectorSubcoreMesh(core_axis_name="c", subcore_axis_name="s")

    @functools.partial(
        pl.kernel,
        out_type=jax.ShapeDtypeStruct((_NC, _N_HALF, _D), jnp.float32),
        mesh=mesh,
        scratch_types=[
            pltpu.VMEM((_CHA,), jnp.int32),
            pltpu.VMEM((_CHB,), jnp.int32),
            pltpu.VMEM((_CHA, _D), jnp.float32),
            pltpu.VMEM((_CHB, _D), jnp.float32),
            pltpu.VMEM_SHARED((_N_HALF + 8, _D), jnp.float32),
            pltpu.SemaphoreType.DMA,
            pltpu.SemaphoreType.DMA,
            pltpu.SemaphoreType.DMA,
            pltpu.SemaphoreType.DMA,
            pltpu.SemaphoreType.DMA,
            pltpu.SemaphoreType.DMA,
        ],
        compiler_params=pltpu.CompilerParams(needs_layout_passes=False),
    )
    def k(h_hbm, col_hbm, zero_hbm, out_hbm, idx_a, idx_b, rows_a, rows_b,
          acc_sh, sem_ia, sem_ib, sem_ra, sem_rb, sem_sa, sem_sb):
        c = lax.axis_index("c")
        s = lax.axis_index("s")
        nbase = c * _N_HALF
        NPAIR = _PER_T // (_CHA + _CHB)
        bufs = ((idx_a, rows_a, _CHA, 0, sem_ia, sem_ra, sem_sa),
                (idx_b, rows_b, _CHB, _CHA, sem_ib, sem_rb, sem_sb))

        def start_dma(pair, bset):
            idx_v, rows_v, ch, off, sem_i, sem_r, _ = bset
            base = s * _PER_T + pair * (_CHA + _CHB) + off
            pltpu.async_copy(col_hbm.at[pl.ds(base, ch)], idx_v, sem_i)
            pltpu.async_copy(h_hbm.at[pl.ds(base, ch)], rows_v, sem_r)

        def wait_dma(pair, bset):
            idx_v, rows_v, ch, off, sem_i, sem_r, _ = bset
            base = s * _PER_T + pair * (_CHA + _CHB) + off
            pltpu.make_async_copy(
                col_hbm.at[pl.ds(base, ch)], idx_v, sem_i).wait()
            pltpu.make_async_copy(
                h_hbm.at[pl.ds(base, ch)], rows_v, sem_r).wait()

        def wait_stream(bset):
            idx_v, rows_v = bset[0], bset[1]
            pltpu.make_async_copy(rows_v, acc_sh.at[idx_v], bset[6]).wait()

        start_dma(0, bufs[0])
        pltpu.sync_copy(zero_hbm, acc_sh.at[pl.ds(s * _ROWS_T, _ROWS_T)])

        @pl.when(s == 0)
        def _():
            pltpu.sync_copy(zero_hbm.at[pl.ds(0, 8)],
                            acc_sh.at[pl.ds(_N_HALF, 8)])

        plsc.subcore_barrier()

        def body(io, carry):
            for b in range(2):
                bset = bufs[b]
                idx_v, rows_v, ch = bset[0], bset[1], bset[2]
                sem_s = bset[6]
                wait_dma(io, bset)

                def tbody(j, c2):
                    c16 = idx_v[pl.ds(j * 16, 16)]
                    loc = c16 - nbase
                    ok = jnp.logical_and(loc >= 0, loc < _N_HALF)
                    idx_v[pl.ds(j * 16, 16)] = jnp.where(ok, loc, _N_HALF)
                    return c2

                lax.fori_loop(0, ch // 16, tbody, 0)
                pltpu.async_copy(rows_v, acc_sh.at[idx_v], sem_s, add=True)
                other = bufs[1 - b]
                if b == 0:
                    @pl.when(io >= 1)
                    def _():
                        wait_stream(other)

                    start_dma(io, other)
                else:
                    @pl.when(io < NPAIR - 1)
                    def _():
                        wait_stream(other)
                        start_dma(io + 1, other)

            return carry

        lax.fori_loop(0, NPAIR, body, 0)
        wait_stream(bufs[0])
        wait_stream(bufs[1])
        plsc.subcore_barrier()
        pltpu.sync_copy(
            acc_sh.at[pl.ds(s * _ROWS_T, _ROWS_T)],
            out_hbm.at[c].at[pl.ds(s * _ROWS_T, _ROWS_T)],
        )

    return k(h, col, zeros_tile)


def _edge_mlp(g, ea, w0x, w0e, b0, w1, b1, w2, b2, w3, b3, w4):
    BE = 2560
    grid = (_E // BE,)

    dot = functools.partial(jnp.dot, preferred_element_type=jnp.float32)
    bf = jnp.bfloat16

    def body(g_ref, e_ref, w0x_r, w0e_r, b0_r, w1_r, b1_r, w2_r, b2_r,
             w3_r, b3_r, w4_r, out_ref):
        g = g_ref[...].astype(bf)
        h = dot(g, w0x_r[...]) + dot(e_ref[...], w0e_r[...]) + b0_r[...]
        h = jnp.maximum(h, 0.0).astype(bf)
        h = jnp.maximum(dot(h, w1_r[...]) + b1_r[...], 0.0).astype(bf)
        h = jnp.maximum(dot(h, w2_r[...]) + b2_r[...], 0.0).astype(bf)
        h = jnp.maximum(dot(h, w3_r[...]) + b3_r[...], 0.0).astype(bf)
        out_ref[...] = dot(h, w4_r[...])

    full = lambda shape: pl.BlockSpec(shape, lambda i: (0,) * len(shape))
    return pl.pallas_call(
        body,
        grid=grid,
        in_specs=[
            pl.BlockSpec((BE, _D), lambda i: (i, 0)),
            pl.BlockSpec((BE, 16), lambda i: (i, 0)),
            full((_D, 128)), full((16, 128)), full((1, 128)),
            full((128, 128)), full((1, 128)),
            full((128, 128)), full((1, 128)),
            full((128, 144)), full((1, 144)),
            full((144, 128)),
        ],
        out_specs=pl.BlockSpec((BE, _D), lambda i: (i, 0)),
        out_shape=jax.ShapeDtypeStruct((_E, _D), jnp.float32),
    )(g, ea, w0x, w0e, b0, w1, b1, w2, b2, w3, b3, w4)


def _node_mlp(x, partials, cnt_parts, ga, w0x, w0g, b0, w1, b1, w2, b2,
              w3, b3):
    BN = 2048
    grid = (_N_PAD // BN,)

    def body(x_ref, p_ref, c_ref, ones_r, ga_r, w0x_r, w0g_r, b0_r, w1_r,
             b1_r, w2_r, b2_r, w3_r, b3_r, out_ref):
        s = p_ref[...]
        cnt = lax.dot_general(c_ref[...], ones_r[...],
                              (((0,), (0,)), ((), ())))
        agg = s * (1.0 / jnp.maximum(cnt, 1.0))
        gb = ga_r[...] @ w0g_r[...] + b0_r[...]
        h = jnp.maximum(x_ref[...] @ w0x_r[...] + agg + gb, 0.0)
        h = jnp.maximum(h @ w1_r[...] + b1_r[...], 0.0)
        h = jnp.maximum(h @ w2_r[...] + b2_r[...], 0.0)
        out_ref[...] = h @ w3_r[...] + b3_r[...]

    full = lambda shape: pl.BlockSpec(shape, lambda i: (0,) * len(shape))
    return pl.pallas_call(
        body,
        grid=grid,
        in_specs=[
            pl.BlockSpec((BN, _D), lambda i: (i, 0)),
            pl.BlockSpec((BN, _D), lambda i: (i, 0)),
            pl.BlockSpec((_NW, BN), lambda i: (0, i)),
            full((_NW, 1)),
            full((1, 16)),
            full((_D, 128)), full((16, 128)), full((1, 128)),
            full((128, 128)), full((1, 128)),
            full((128, 128)), full((1, 128)),
            full((128, 128)), full((1, 128)),
        ],
        out_specs=pl.BlockSpec((BN, 128), lambda i: (i, 0)),
        out_shape=jax.ShapeDtypeStruct((_N_PAD, 128), jnp.float32),
    )(x, partials, cnt_parts, jnp.ones((_NW, 1), jnp.float32), ga, w0x,
      w0g, b0, w1, b1, w2, b2, w3, b3)


def kernel(x, edge_index, edge_attr, global_attr,
           W10, b10, W11, b11, W12, b12, W13, b13,
           W20, b20, W21, b21, W22, b22, W23, b23):
    row = edge_index[0]
    col = edge_index[1]

    w10x = W10[:_D]
    w10e = W10[_D:]
    w20x = W20[:_D]
    w20a = W20[_D:_D + 144]
    w20g = W20[_D + 144:]

    bf = jnp.bfloat16
    gathered, cnt_flat = _sc_gather_count(x, row, col)
    cnt_parts = cnt_flat.reshape(_NW, _N_PAD)
    h5 = _edge_mlp(gathered, edge_attr.astype(bf),
                   w10x.astype(bf), w10e.astype(bf), b10.reshape(1, -1),
                   W11.astype(bf), b11.reshape(1, -1),
                   W12.astype(bf), b12.reshape(1, -1),
                   W13.astype(bf), b13.reshape(1, -1),
                   w20a.astype(bf))
    zeros_tile = jnp.zeros((_ROWS_T, _D), jnp.float32)
    sums = _sc_scatter(h5, col, zeros_tile).reshape(_N_PAD, _D)
    x_pad = jnp.concatenate(
        [x, jnp.zeros((_N_PAD - _N, _D), jnp.float32)], axis=0)
    out = _node_mlp(x_pad, sums, cnt_parts, global_attr,
                    w20x, w20g, b20.reshape(1, -1),
                    W21, b21.reshape(1, -1),
                    W22, b22.reshape(1, -1),
                    W23, b23.reshape(1, -1))
    return out[:_N]
